# baseline (speedup 1.0000x reference)
import functools

import jax
import jax.numpy as jnp
from jax import lax
from jax.experimental import pallas as pl
from jax.experimental.pallas import tpu as pltpu
from jax.experimental.pallas import tpu_sc as plsc

NUM_CORES = 2
NUM_SUBCORES = 16
NUM_WORKERS = NUM_CORES * NUM_SUBCORES

CHUNK = 256
STREAM = 128
NS = CHUNK // STREAM


@jax.jit
def _sc_gather(ids2, table2):
    b_total = ids2.shape[0] * STREAM
    b_per_w = b_total // NUM_WORKERS
    id_rows_w = b_per_w // STREAM
    out_rows_w = b_per_w // 2
    n_pairs = b_per_w // CHUNK // 2
    mesh = plsc.VectorSubcoreMesh(core_axis_name="c", subcore_axis_name="s")

    @functools.partial(
        pl.kernel,
        mesh=mesh,
        out_type=jax.ShapeDtypeStruct((b_total // 2, 128), jnp.float32),
        scratch_types=[
            pltpu.VMEM((id_rows_w, STREAM), jnp.int32),
            pltpu.VMEM((2, NS, STREAM), jnp.int32),
            pltpu.VMEM((CHUNK, 128), jnp.float32),
            pltpu.VMEM((CHUNK, 128), jnp.float32),
            pltpu.VMEM((CHUNK // 2, 128), jnp.float32),
            pltpu.VMEM((CHUNK // 2, 128), jnp.float32),
            pltpu.SemaphoreType.DMA,
            pltpu.SemaphoreType.DMA,
            pltpu.SemaphoreType.DMA,
            pltpu.SemaphoreType.DMA,
        ],
        compiler_params=pltpu.CompilerParams(
            use_tc_tiling_on_sc=True, needs_layout_passes=False),
    )
    def k(ids_hbm, table_hbm, out_hbm, ids_v, pidx_v, gath0, gath1,
          comp0, comp1, g0, g1, o0, o1):
        wid = lax.axis_index("s") * NUM_CORES + lax.axis_index("c")
        out_base = wid * out_rows_w
        pltpu.sync_copy(ids_hbm.at[pl.ds(wid * id_rows_w, id_rows_w)], ids_v)

        iota16 = lax.iota(jnp.int32, 16)
        alt64 = (iota16 & 1) * 64

        def prep_idx(slot, ch):
            for j in range(NS):
                for kk in range(STREAM // 16):
                    v = ids_v[NS * ch + j, pl.ds(kk * 16, 16)]
                    pidx_v[slot, j, pl.ds(kk * 16, 16)] = v >> 1

        def fire(gath, slot, ch, gsem):
            for j in range(NS):
                pltpu.async_copy(
                    table_hbm.at[pidx_v.at[slot, j]],
                    gath.at[pl.ds(j * STREAM, STREAM)],
                    gsem,
                )

        def drain(gath, slot, ch, gsem):
            for j in range(NS):
                pltpu.make_async_copy(
                    table_hbm.at[pidx_v.at[slot, j]],
                    gath.at[pl.ds(j * STREAM, STREAM)],
                    gsem,
                ).wait()

        def select(gath, comp, ch):
            def group(g, carry):
                row_vec = iota16 + g * 16
                row2_vec = lax.shift_right_logical(row_vec, 1)
                ids_vec = ids_v[NS * ch + g // 8, pl.ds((g % 8) * 16, 16)]
                par64 = (ids_vec & 1) * 64
                for c in range(64):
                    v = plsc.load_gather(gath, [row_vec, par64 + c])
                    plsc.store_scatter(comp, [row2_vec, alt64 + c], v)
                return carry

            lax.fori_loop(0, CHUNK // 16, group, 0)

        def store(comp, ch, osem):
            pltpu.async_copy(
                comp, out_hbm.at[pl.ds(out_base + ch * (CHUNK // 2), CHUNK // 2)],
                osem,
            )

        def wait_store(comp, ch, osem):
            pltpu.make_async_copy(
                comp, out_hbm.at[pl.ds(out_base + ch * (CHUNK // 2), CHUNK // 2)],
                osem,
            ).wait()

        def body(i, carry):
            c0 = 2 * i
            c1 = 2 * i + 1

            @pl.when(i > 0)
            def _():
                wait_store(comp0, c0 - 2, o0)

            prep_idx(0, c0)
            fire(gath0, 0, c0, g0)

            @pl.when(i > 0)
            def _():
                wait_store(comp1, c1 - 2, o1)

            prep_idx(1, c1)
            fire(gath1, 1, c1, g1)

            drain(gath0, 0, c0, g0)
            select(gath0, comp0, c0)
            store(comp0, c0, o0)

            drain(gath1, 1, c1, g1)
            select(gath1, comp1, c1)
            store(comp1, c1, o1)
            return carry

        lax.fori_loop(0, n_pairs, body, 0)
        wait_store(comp0, 2 * n_pairs - 2, o0)
        wait_store(comp1, 2 * n_pairs - 1, o1)

    return k(ids2, table2)


def kernel(token_ids, embeddings):
    b_total = token_ids.shape[0] * token_ids.shape[1]
    d = embeddings.shape[1]
    flat = token_ids.reshape(b_total // STREAM, STREAM).astype(jnp.int32)
    table2 = embeddings.reshape(embeddings.shape[0] // 2, 2 * d)
    out2 = _sc_gather(flat, table2)
    return out2.reshape(token_ids.shape + (d,))


# speedup vs baseline: 2.2746x; 2.2746x over previous
import functools

import jax
import jax.numpy as jnp
from jax import lax
from jax.experimental import pallas as pl
from jax.experimental.pallas import tpu as pltpu
from jax.experimental.pallas import tpu_sc as plsc

NUM_CORES = 2
NUM_SUBCORES = 16
NUM_WORKERS = NUM_CORES * NUM_SUBCORES

CHUNK = 640
STREAM = 128


@functools.partial(jax.jit, static_argnums=(2, 3))
def _sc_gather(flat_ids, table, b_total, d):
    b_per_w = b_total // NUM_WORKERS
    n_chunks = b_per_w // CHUNK
    n_pairs = n_chunks // 2
    n_streams = CHUNK // STREAM
    mesh = plsc.VectorSubcoreMesh(core_axis_name="c", subcore_axis_name="s")

    @functools.partial(
        pl.kernel,
        mesh=mesh,
        out_type=jax.ShapeDtypeStruct((b_total, d), jnp.float32),
        scratch_types=[
            pltpu.VMEM((b_per_w // STREAM, STREAM), jnp.int32),
            pltpu.VMEM((CHUNK, d), jnp.float32),
            pltpu.VMEM((CHUNK, d), jnp.float32),
            pltpu.SemaphoreType.DMA,
            pltpu.SemaphoreType.DMA,
            pltpu.SemaphoreType.DMA,
            pltpu.SemaphoreType.DMA,
        ],
        compiler_params=pltpu.CompilerParams(use_tc_tiling_on_sc=False),
    )
    def k(ids_hbm, table_hbm, out_hbm, ids_v, rows0, rows1, g0, g1, o0, o1):
        wid = lax.axis_index("s") * NUM_CORES + lax.axis_index("c")
        base = wid * b_per_w
        rows_per_w = b_per_w // STREAM
        pltpu.sync_copy(ids_hbm.at[pl.ds(wid * rows_per_w, rows_per_w)], ids_v)

        def fire(slot, ch, gsem):
            for j in range(n_streams):
                pltpu.async_copy(
                    table_hbm.at[ids_v.at[ch * n_streams + j]],
                    slot.at[pl.ds(j * STREAM, STREAM)],
                    gsem,
                )

        def drain(slot, ch, gsem):
            for j in range(n_streams):
                pltpu.make_async_copy(
                    table_hbm.at[ids_v.at[ch * n_streams + j]],
                    slot.at[pl.ds(j * STREAM, STREAM)],
                    gsem,
                ).wait()

        def store(slot, ch, osem):
            pltpu.async_copy(
                slot, out_hbm.at[pl.ds(base + ch * CHUNK, CHUNK)], osem
            )

        def wait_store(slot, ch, osem):
            pltpu.make_async_copy(
                slot, out_hbm.at[pl.ds(base + ch * CHUNK, CHUNK)], osem
            ).wait()

        def body(i, carry):
            c0 = 2 * i
            c1 = 2 * i + 1

            @pl.when(i > 0)
            def _():
                wait_store(rows0, c0 - 2, o0)

            fire(rows0, c0, g0)

            @pl.when(i > 0)
            def _():
                wait_store(rows1, c1 - 2, o1)

            fire(rows1, c1, g1)
            drain(rows0, c0, g0)
            store(rows0, c0, o0)
            drain(rows1, c1, g1)
            store(rows1, c1, o1)
            return carry

        lax.fori_loop(0, n_pairs, body, 0)
        wait_store(rows0, n_chunks - 2, o0)
        wait_store(rows1, n_chunks - 1, o1)

    return k(flat_ids, table)


def kernel(token_ids, embeddings):
    b, s = token_ids.shape
    d = embeddings.shape[1]
    n_split = 2
    sp = s // n_split
    parts = []
    for p in range(n_split):
        bt = b * sp
        flat = token_ids[:, p * sp:(p + 1) * sp].reshape(
            bt // STREAM, STREAM).astype(jnp.int32)
        outp = _sc_gather(flat, embeddings, bt, d)
        parts.append(outp.reshape(b, sp, d))
    return jnp.concatenate(parts, axis=1)


# speedup vs baseline: 2.4746x; 1.0879x over previous
import functools

import jax
import jax.numpy as jnp
from jax import lax
from jax.experimental import pallas as pl
from jax.experimental.pallas import tpu as pltpu
from jax.experimental.pallas import tpu_sc as plsc

NUM_CORES = 2
NUM_SUBCORES = 16
NUM_WORKERS = NUM_CORES * NUM_SUBCORES

CHUNK = 640
STREAM = 128


@functools.partial(jax.jit, static_argnums=(2, 3))
def _sc_gather(flat_ids, table, b_total, d):
    b_per_w = b_total // NUM_WORKERS
    n_chunks = b_per_w // CHUNK
    n_pairs = n_chunks // 2
    n_streams = CHUNK // STREAM
    mesh = plsc.VectorSubcoreMesh(core_axis_name="c", subcore_axis_name="s")

    @functools.partial(
        pl.kernel,
        mesh=mesh,
        out_type=jax.ShapeDtypeStruct((b_total, d), jnp.float32),
        scratch_types=[
            pltpu.VMEM((b_per_w // STREAM, STREAM), jnp.int32),
            pltpu.VMEM((CHUNK, d), jnp.float32),
            pltpu.VMEM((CHUNK, d), jnp.float32),
            pltpu.SemaphoreType.DMA,
            pltpu.SemaphoreType.DMA,
            pltpu.SemaphoreType.DMA,
            pltpu.SemaphoreType.DMA,
        ],
        compiler_params=pltpu.CompilerParams(use_tc_tiling_on_sc=False),
    )
    def k(ids_hbm, table_hbm, out_hbm, ids_v, rows0, rows1, g0, g1, o0, o1):
        wid = lax.axis_index("s") * NUM_CORES + lax.axis_index("c")
        base = wid * b_per_w
        rows_per_w = b_per_w // STREAM
        pltpu.sync_copy(ids_hbm.at[pl.ds(wid * rows_per_w, rows_per_w)], ids_v)

        def fire(slot, ch, gsem):
            for j in range(n_streams):
                pltpu.async_copy(
                    table_hbm.at[ids_v.at[ch * n_streams + j]],
                    slot.at[pl.ds(j * STREAM, STREAM)],
                    gsem,
                )

        def drain(slot, ch, gsem):
            for j in range(n_streams):
                pltpu.make_async_copy(
                    table_hbm.at[ids_v.at[ch * n_streams + j]],
                    slot.at[pl.ds(j * STREAM, STREAM)],
                    gsem,
                ).wait()

        def store(slot, ch, osem):
            pltpu.async_copy(
                slot, out_hbm.at[pl.ds(base + ch * CHUNK, CHUNK)], osem
            )

        def wait_store(slot, ch, osem):
            pltpu.make_async_copy(
                slot, out_hbm.at[pl.ds(base + ch * CHUNK, CHUNK)], osem
            ).wait()

        def body(i, carry):
            c0 = 2 * i
            c1 = 2 * i + 1

            @pl.when(i > 0)
            def _():
                wait_store(rows0, c0 - 2, o0)

            fire(rows0, c0, g0)

            @pl.when(i > 0)
            def _():
                wait_store(rows1, c1 - 2, o1)

            fire(rows1, c1, g1)
            drain(rows0, c0, g0)
            store(rows0, c0, o0)
            drain(rows1, c1, g1)
            store(rows1, c1, o1)
            return carry

        lax.fori_loop(0, n_pairs, body, 0)
        wait_store(rows0, n_chunks - 2, o0)
        wait_store(rows1, n_chunks - 1, o1)

    return k(flat_ids, table)


def kernel(token_ids, embeddings):
    b_total = token_ids.shape[0] * token_ids.shape[1]
    d = embeddings.shape[1]
    flat = token_ids.reshape(b_total // STREAM, STREAM).astype(jnp.int32)
    out = _sc_gather(flat, embeddings, b_total, d)
    return out.reshape(token_ids.shape + (d,))
